# baseline (speedup 1.0000x reference)
import functools

import jax
import jax.numpy as jnp
from jax import lax
from jax.experimental import pallas as pl
from jax.experimental.pallas import tpu as pltpu
from jax.experimental.pallas import tpu_sc as plsc

_NC = 2
_NS = 16
_NW = _NC * _NS
_L = 16


def _make_sc_assign(E, N):
    per = E // _NW
    C = 5000 if per % 5000 == 0 else per
    n_chunks = per // C
    n_vec = (C + _L - 1) // _L
    c_pad = n_vec * _L

    mesh = plsc.VectorSubcoreMesh(core_axis_name="c", subcore_axis_name="s")

    idx_buf = pltpu.VMEM((c_pad,), jnp.int32)
    col_buf = pltpu.VMEM((N,), jnp.float32)

    @functools.partial(
        pl.kernel,
        mesh=mesh,
        out_type=(jax.ShapeDtypeStruct((E,), jnp.int32),
                  jax.ShapeDtypeStruct((2 * E,), jnp.int32)),
        compiler_params=pltpu.CompilerParams(needs_layout_passes=False),
        scratch_types=[
            idx_buf, idx_buf,
            col_buf, col_buf,
            pltpu.VMEM((c_pad,), jnp.int32),
            pltpu.VMEM((2 * c_pad,), jnp.int32),
            pltpu.VMEM((6 * _L,), jnp.float32),
            pltpu.SemaphoreType.DMA,
            pltpu.SemaphoreType.DMA,
        ],
    )
    def sc_assign(rx_hbm, ry_hbm, rz_hbm, qx_hbm, qy_hbm, qz_hbm,
                  eref_hbm, equery_hbm, kp_hbm, part_hbm, pair_hbm,
                  er_v, eq_v, rcol, qcol, out_v, pair_v, kp_v,
                  sem_r, sem_q):
        wid = (lax.axis_index("s").astype(jnp.int32) * jnp.int32(_NC)
               + lax.axis_index("c").astype(jnp.int32))
        tile_base = wid * jnp.int32(per)

        pltpu.sync_copy(kp_hbm, kp_v)
        zero = jnp.full((_L,), 0, jnp.int32)
        lanes = lax.iota(jnp.int32, _L)
        if c_pad > C:
            er_v[pl.ds(jnp.int32(c_pad - _L), _L)] = zero
            eq_v[pl.ds(jnp.int32(c_pad - _L), _L)] = zero

        def zbody(i, carry):
            base = i * jnp.int32(4 * _L)
            for u in range(4):
                pair_v[pl.ds(base + jnp.int32(u * _L), _L)] = zero
            return carry

        lax.fori_loop(jnp.int32(0), jnp.int32((2 * n_vec) // 4), zbody, 0,
                      unroll=False)
        for u in range((2 * n_vec) % 4):
            pair_v[pl.ds(jnp.int32(((2 * n_vec) // 4 * 4 + u) * _L), _L)] \
                = zero

        axes = [
            (0, 9, rx_hbm, qx_hbm),
            (1, 3, ry_hbm, qy_hbm),
            (2, 1, rz_hbm, qz_hbm),
        ]
        for a, w, rcol_hbm, qcol_hbm in axes:
            cp_r = pltpu.async_copy(rcol_hbm, rcol, sem_r)
            cp_q = pltpu.async_copy(qcol_hbm, qcol, sem_q)
            cp_r.wait()
            cp_q.wait()
            hv = kp_v[pl.ds(a * _L, _L)]
            nhv = kp_v[pl.ds((3 + a) * _L, _L)]
            wv = jnp.full((_L,), w, jnp.int32)
            last = a == 2

            for j in range(n_chunks):
                base_e = tile_base + jnp.int32(j * C)
                pltpu.sync_copy(eref_hbm.at[pl.ds(base_e, C)],
                                er_v.at[pl.ds(0, C)])
                pltpu.sync_copy(equery_hbm.at[pl.ds(base_e, C)],
                                eq_v.at[pl.ds(0, C)])
                if a > 0:
                    pltpu.sync_copy(part_hbm.at[pl.ds(base_e, C)],
                                    out_v.at[pl.ds(0, C)])

                def vec_step(off):
                    sl = pl.ds(off, _L)
                    t = (plsc.load_gather(rcol, [er_v[sl]])
                         - plsc.load_gather(qcol, [eq_v[sl]]))
                    contrib = (jnp.where(t > nhv, wv, zero)
                               + jnp.where(t > hv, wv, zero))
                    if a > 0:
                        contrib = contrib + out_v[sl]
                    if last:
                        pidx = (jnp.full((_L,), off, jnp.int32) + lanes
                                ) * jnp.int32(2)
                        plsc.store_scatter(pair_v, [pidx], contrib)
                    else:
                        out_v[sl] = contrib

                def body(i, carry):
                    base = i * jnp.int32(4 * _L)
                    for u in range(4):
                        vec_step(base + jnp.int32(u * _L))
                    return carry

                lax.fori_loop(jnp.int32(0), jnp.int32(n_vec // 4), body, 0,
                              unroll=False)
                for u in range(n_vec % 4):
                    vec_step(jnp.int32((n_vec // 4 * 4 + u) * _L))
                if last:
                    pltpu.sync_copy(pair_v.at[pl.ds(0, 2 * C)],
                                    pair_hbm.at[pl.ds(base_e * jnp.int32(2),
                                                      2 * C)])
                else:
                    pltpu.sync_copy(out_v.at[pl.ds(0, C)],
                                    part_hbm.at[pl.ds(base_e, C)])

    return sc_assign


def kernel(ref_bxyz, query_bxyz, e_ref, e_query, kernel_pos):
    E = e_ref.shape[0]
    N = ref_bxyz.shape[0]
    er = e_ref.astype(jnp.int32)
    eq = e_query.astype(jnp.int32)
    rx, ry, rz = ref_bxyz[:, 1], ref_bxyz[:, 2], ref_bxyz[:, 3]
    qx, qy, qz = query_bxyz[:, 1], query_bxyz[:, 2], query_bxyz[:, 3]
    h = kernel_pos[26, :].astype(jnp.float32) * jnp.float32(0.5)
    kp_pad = jnp.concatenate([jnp.repeat(h, _L), jnp.repeat(-h, _L)])
    _, pairs = _make_sc_assign(E, N)(rx, ry, rz, qx, qy, qz, er, eq, kp_pad)
    return lax.bitcast_convert_type(pairs.reshape(E, 2), jnp.int64)


# speedup vs baseline: 4.5207x; 4.5207x over previous
import functools

import jax
import jax.numpy as jnp
from jax import lax
from jax.experimental import pallas as pl
from jax.experimental.pallas import tpu as pltpu
from jax.experimental.pallas import tpu_sc as plsc

_NC = 2
_NS = 16
_NW = _NC * _NS
_L = 16


def _make_sc_assign(E, N):
    per = E // _NW
    C = 5000 if per % 5000 == 0 else per
    n_chunks = per // C
    n_vec = (C + _L - 1) // _L
    c_pad = n_vec * _L

    mesh = plsc.VectorSubcoreMesh(core_axis_name="c", subcore_axis_name="s")

    idx_buf = pltpu.VMEM((c_pad,), jnp.int32)
    col_buf = pltpu.VMEM((N,), jnp.float32)

    @functools.partial(
        pl.kernel,
        mesh=mesh,
        out_type=jax.ShapeDtypeStruct((E,), jnp.int32),
        compiler_params=pltpu.CompilerParams(needs_layout_passes=False),
        scratch_types=[
            idx_buf, idx_buf,
            col_buf, col_buf,
            pltpu.VMEM((c_pad,), jnp.int32),
            pltpu.VMEM((6 * _L,), jnp.float32),
            pltpu.SemaphoreType.DMA,
            pltpu.SemaphoreType.DMA,
        ],
    )
    def sc_assign(rx_hbm, ry_hbm, rz_hbm, qx_hbm, qy_hbm, qz_hbm,
                  eref_hbm, equery_hbm, kp_hbm, out_hbm,
                  er_v, eq_v, rcol, qcol, out_v, kp_v,
                  sem_r, sem_q):
        wid = (lax.axis_index("s").astype(jnp.int32) * jnp.int32(_NC)
               + lax.axis_index("c").astype(jnp.int32))
        tile_base = wid * jnp.int32(per)

        pltpu.sync_copy(kp_hbm, kp_v)
        zero = jnp.full((_L,), 0, jnp.int32)
        lanes = lax.iota(jnp.int32, _L)
        if c_pad > C:
            er_v[pl.ds(jnp.int32(c_pad - _L), _L)] = zero
            eq_v[pl.ds(jnp.int32(c_pad - _L), _L)] = zero

        axes = [
            (0, 9, rx_hbm, qx_hbm),
            (1, 3, ry_hbm, qy_hbm),
            (2, 1, rz_hbm, qz_hbm),
        ]
        for a, w, rcol_hbm, qcol_hbm in axes:
            cp_r = pltpu.async_copy(rcol_hbm, rcol, sem_r)
            cp_q = pltpu.async_copy(qcol_hbm, qcol, sem_q)
            cp_r.wait()
            cp_q.wait()
            hv = kp_v[pl.ds(a * _L, _L)]
            nhv = kp_v[pl.ds((3 + a) * _L, _L)]
            wv = jnp.full((_L,), w, jnp.int32)

            for j in range(n_chunks):
                base_e = tile_base + jnp.int32(j * C)
                pltpu.sync_copy(eref_hbm.at[pl.ds(base_e, C)],
                                er_v.at[pl.ds(0, C)])
                pltpu.sync_copy(equery_hbm.at[pl.ds(base_e, C)],
                                eq_v.at[pl.ds(0, C)])
                if a > 0:
                    pltpu.sync_copy(out_hbm.at[pl.ds(base_e, C)],
                                    out_v.at[pl.ds(0, C)])

                def vec_step(off):
                    sl = pl.ds(off, _L)
                    t = (plsc.load_gather(rcol, [er_v[sl]])
                         - plsc.load_gather(qcol, [eq_v[sl]]))
                    contrib = (jnp.where(t > nhv, wv, zero)
                               + jnp.where(t > hv, wv, zero))
                    if a > 0:
                        contrib = contrib + out_v[sl]
                    out_v[sl] = contrib

                def body(i, carry):
                    base = i * jnp.int32(4 * _L)
                    for u in range(4):
                        vec_step(base + jnp.int32(u * _L))
                    return carry

                lax.fori_loop(jnp.int32(0), jnp.int32(n_vec // 4), body, 0,
                              unroll=False)
                for u in range(n_vec % 4):
                    vec_step(jnp.int32((n_vec // 4 * 4 + u) * _L))
                pltpu.sync_copy(out_v.at[pl.ds(0, C)],
                                out_hbm.at[pl.ds(base_e, C)])

    return sc_assign


def kernel(ref_bxyz, query_bxyz, e_ref, e_query, kernel_pos):
    E = e_ref.shape[0]
    N = ref_bxyz.shape[0]
    er = e_ref.astype(jnp.int32)
    eq = e_query.astype(jnp.int32)
    rx, ry, rz = ref_bxyz[:, 1], ref_bxyz[:, 2], ref_bxyz[:, 3]
    qx, qy, qz = query_bxyz[:, 1], query_bxyz[:, 2], query_bxyz[:, 3]
    h = kernel_pos[26, :].astype(jnp.float32) * jnp.float32(0.5)
    kp_pad = jnp.concatenate([jnp.repeat(h, _L), jnp.repeat(-h, _L)])
    out32 = _make_sc_assign(E, N)(rx, ry, rz, qx, qy, qz, er, eq, kp_pad)
    return out32.astype(jnp.int64)


# speedup vs baseline: 5.6871x; 1.2580x over previous
import functools

import jax
import jax.numpy as jnp
from jax import lax
from jax.experimental import pallas as pl
from jax.experimental.pallas import tpu as pltpu
from jax.experimental.pallas import tpu_sc as plsc

_NC = 2
_NS = 16
_NW = _NC * _NS
_L = 16


def _make_sc_assign(E, N):
    per = E // _NW
    C = 5000 if per % 5000 == 0 else per
    n_chunks = per // C
    n_vec = (C + _L - 1) // _L
    c_pad = n_vec * _L

    mesh = plsc.VectorSubcoreMesh(core_axis_name="c", subcore_axis_name="s")

    idx_buf = pltpu.VMEM((c_pad,), jnp.int32)
    col_buf = pltpu.VMEM((N,), jnp.float32)

    @functools.partial(
        pl.kernel,
        mesh=mesh,
        out_type=jax.ShapeDtypeStruct((E,), jnp.int32),
        compiler_params=pltpu.CompilerParams(needs_layout_passes=False),
        scratch_types=[
            idx_buf, idx_buf,
            idx_buf, idx_buf,
            col_buf, col_buf,
            pltpu.VMEM((c_pad,), jnp.int32),
            pltpu.VMEM((c_pad,), jnp.int32),
            pltpu.VMEM((6 * _L,), jnp.float32),
            pltpu.SemaphoreType.DMA,
            pltpu.SemaphoreType.DMA,
            pltpu.SemaphoreType.DMA,
            pltpu.SemaphoreType.DMA,
            pltpu.SemaphoreType.DMA,
        ],
    )
    def sc_assign(rx_hbm, ry_hbm, rz_hbm, qx_hbm, qy_hbm, qz_hbm,
                  eref_hbm, equery_hbm, kp_hbm, out_hbm,
                  er0, er1, eq0, eq1, rcol, qcol, out0, out1, kp_v,
                  sem_c, isem0, isem1, osem0, osem1):
        wid = (lax.axis_index("s").astype(jnp.int32) * jnp.int32(_NC)
               + lax.axis_index("c").astype(jnp.int32))
        tile_base = wid * jnp.int32(per)

        pltpu.sync_copy(kp_hbm, kp_v)
        zero = jnp.full((_L,), 0, jnp.int32)
        if c_pad > C:
            tail = pl.ds(jnp.int32(c_pad - _L), _L)
            er0[tail] = zero
            er1[tail] = zero
            eq0[tail] = zero
            eq1[tail] = zero

        ibufs = [(er0, eq0, out0, isem0, osem0),
                 (er1, eq1, out1, isem1, osem1)]

        axes = [
            (0, 9, rx_hbm, qx_hbm),
            (1, 3, ry_hbm, qy_hbm),
            (2, 1, rz_hbm, qz_hbm),
        ]
        for a, w, rcol_hbm, qcol_hbm in axes:
            cp_r = pltpu.async_copy(rcol_hbm, rcol, sem_c)
            cp_q = pltpu.async_copy(qcol_hbm, qcol, sem_c)

            def stage(j, owrite_cp):
                er, eq, out_v, isem, _ = ibufs[j % 2]
                base_e = tile_base + jnp.int32(j * C)
                if owrite_cp is not None:
                    owrite_cp.wait()
                cps = [
                    pltpu.async_copy(eref_hbm.at[pl.ds(base_e, C)],
                                     er.at[pl.ds(0, C)], isem),
                    pltpu.async_copy(equery_hbm.at[pl.ds(base_e, C)],
                                     eq.at[pl.ds(0, C)], isem),
                ]
                if a > 0:
                    cps.append(
                        pltpu.async_copy(out_hbm.at[pl.ds(base_e, C)],
                                         out_v.at[pl.ds(0, C)], isem))
                return cps

            pending = {0: stage(0, None)}
            owrites = {}
            cp_r.wait()
            cp_q.wait()

            hv = kp_v[pl.ds(a * _L, _L)]
            nhv = kp_v[pl.ds((3 + a) * _L, _L)]
            wv = jnp.full((_L,), w, jnp.int32)

            for j in range(n_chunks):
                if j + 1 < n_chunks:
                    pending[j + 1] = stage(j + 1, owrites.pop(j - 1, None))
                for cp in pending.pop(j):
                    cp.wait()
                er, eq, out_v, _, osem = ibufs[j % 2]

                def body(i, carry):
                    sl = pl.ds(i * jnp.int32(_L), _L)
                    t = (plsc.load_gather(rcol, [er[sl]])
                         - plsc.load_gather(qcol, [eq[sl]]))
                    contrib = (jnp.where(t > nhv, wv, zero)
                               + jnp.where(t > hv, wv, zero))
                    if a > 0:
                        contrib = contrib + out_v[sl]
                    out_v[sl] = contrib
                    return carry

                lax.fori_loop(jnp.int32(0), jnp.int32(n_vec), body, 0,
                              unroll=False)
                base_e = tile_base + jnp.int32(j * C)
                owrites[j] = pltpu.async_copy(
                    out_v.at[pl.ds(0, C)], out_hbm.at[pl.ds(base_e, C)],
                    osem)
            for cp in owrites.values():
                cp.wait()

    return sc_assign


def kernel(ref_bxyz, query_bxyz, e_ref, e_query, kernel_pos):
    E = e_ref.shape[0]
    N = ref_bxyz.shape[0]
    er = e_ref.astype(jnp.int32)
    eq = e_query.astype(jnp.int32)
    rx, ry, rz = ref_bxyz[:, 1], ref_bxyz[:, 2], ref_bxyz[:, 3]
    qx, qy, qz = query_bxyz[:, 1], query_bxyz[:, 2], query_bxyz[:, 3]
    h = kernel_pos[26, :].astype(jnp.float32) * jnp.float32(0.5)
    kp_pad = jnp.concatenate([jnp.repeat(h, _L), jnp.repeat(-h, _L)])
    out32 = _make_sc_assign(E, N)(rx, ry, rz, qx, qy, qz, er, eq, kp_pad)
    return out32.astype(jnp.int64)
